# baseline (speedup 1.0000x reference)
import functools

import jax
import jax.numpy as jnp
from jax import lax
from jax.experimental import pallas as pl
from jax.experimental.pallas import tpu as pltpu
from jax.experimental.pallas import tpu_sc as plsc

_N = 10000
_E = 320000
_D = 128
_K = 3

_NC = 2
_NS = 16
_NW = _NC * _NS
_EP = _E // _NW
_CH = 128
_NCH = 80
_EPP = _NCH * _CH
_SB = 8
_NSB = _NCH // _SB
_NR = 10240
_RPT = _NR // _NS

_BLK = 512
_NB = _NR // _BLK

@functools.cache
def _sc_kernels():
    mesh = plsc.VectorSubcoreMesh(
        core_axis_name="c", subcore_axis_name="s",
        num_cores=_NC, num_subcores=_NS)

    HR = 128
    RR = HR // _NS

    @functools.partial(
        pl.kernel,
        out_type=jax.ShapeDtypeStruct((_NC, 2, HR, 128), jnp.float32),
        mesh=mesh,
        compiler_params=pltpu.CompilerParams(needs_layout_passes=False),
        scratch_types=[
            pltpu.VMEM((_NCH, _CH), jnp.int32),
            pltpu.VMEM((_NCH, _CH), jnp.int32),
            pltpu.VMEM((HR, 128), jnp.float32),
            pltpu.VMEM((_NS, RR, 128), jnp.float32),
            pltpu.VMEM_SHARED((_NS, HR, 128), jnp.float32),
        ],
    )
    def sc_degrees(e_hbm, out_hbm, sidx, didx, hist, rbuf, part):
        c = lax.axis_index("c")
        s = lax.axis_index("s")
        w = c * _NS + s
        pltpu.sync_copy(e_hbm.at[0, w], sidx)
        pltpu.sync_copy(e_hbm.at[1, w], didx)
        ones = jnp.ones((16,), jnp.float32)
        for kind in range(2):
            idxbuf = sidx if kind == 0 else didx

            def zh(i, carry):
                for k16 in range(128 // 16):
                    hist[i, pl.ds(k16 * 16, 16)] = jnp.zeros((16,), jnp.float32)
                return carry

            lax.fori_loop(0, HR, zh, 0)

            def hloop(j, carry):
                for k16 in range(_CH // 16):
                    v = idxbuf[j, pl.ds(k16 * 16, 16)]
                    hi = jax.lax.shift_right_logical(v, 7)
                    lo = jax.lax.bitwise_and(v, 127)
                    plsc.addupdate_scatter(hist, [hi, lo], ones)
                return carry

            lax.fori_loop(0, _NCH, hloop, 0)

            pltpu.sync_copy(hist, part.at[s])
            plsc.subcore_barrier()
            for p in range(_NS):
                pltpu.sync_copy(part.at[p, pl.ds(s * RR, RR)], rbuf.at[p])

            def red(k, carry):
                for k16 in range(128 // 16):
                    acc = rbuf[0, k, pl.ds(k16 * 16, 16)]
                    for p in range(1, _NS):
                        acc = acc + rbuf[p, k, pl.ds(k16 * 16, 16)]
                    hist[k, pl.ds(k16 * 16, 16)] = acc
                return carry

            lax.fori_loop(0, RR, red, 0)
            pltpu.sync_copy(hist.at[pl.ds(0, RR)],
                            out_hbm.at[c, kind, pl.ds(s * RR, RR)])
            plsc.subcore_barrier()

    @functools.partial(
        pl.kernel,
        out_type=jax.ShapeDtypeStruct((_NC, _N, _D), jnp.float32),
        mesh=mesh,
        compiler_params=pltpu.CompilerParams(needs_layout_passes=False),
        scratch_types=[
            pltpu.VMEM((_NCH, _CH), jnp.int32),
            pltpu.VMEM((_NCH, _CH), jnp.int32),
            pltpu.VMEM((_CH, _D), jnp.float32),
            pltpu.VMEM_SHARED((_NR, _D), jnp.float32),
            pltpu.SemaphoreType.DMA,
        ],
    )
    def sc_segsum(e_hbm, xs_hbm, out_hbm, sidx, didx, gbuf, acc, sem):
        c = lax.axis_index("c")
        s = lax.axis_index("s")
        w = c * _NS + s

        def fill(i, carry):
            for k16 in range(_D // 16):
                gbuf[i, pl.ds(k16 * 16, 16)] = jnp.zeros((16,), jnp.float32)
            return carry

        lax.fori_loop(0, _CH, fill, 0)

        def zero(i, carry):
            pltpu.sync_copy(gbuf, acc.at[pl.ds(s * _RPT + i * _CH, _CH)])
            return carry

        lax.fori_loop(0, _RPT // _CH, zero, 0)

        pltpu.sync_copy(e_hbm.at[0, w], sidx)
        pltpu.sync_copy(e_hbm.at[1, w], didx)
        plsc.subcore_barrier()

        def step(j, carry):
            pltpu.async_copy(xs_hbm.at[sidx.at[j]], gbuf, sem).wait()
            pltpu.sync_copy(gbuf, acc.at[didx.at[j]], add=True)
            return carry

        lax.fori_loop(0, _NCH, step, 0)
        plsc.subcore_barrier()

        @pl.when(s < _NS - 1)
        def _():
            pltpu.sync_copy(acc.at[pl.ds(s * _RPT, _RPT)],
                            out_hbm.at[c, pl.ds(s * _RPT, _RPT)])

        @pl.when(s == _NS - 1)
        def _():
            tail = _N - (_NS - 1) * _RPT
            pltpu.sync_copy(acc.at[pl.ds((_NS - 1) * _RPT, tail)],
                            out_hbm.at[c, pl.ds((_NS - 1) * _RPT, tail)])

    return sc_degrees, sc_segsum


def _scale_body(x_ref, d_ref, xs_ref, so_ref, di_ref):
    dv = d_ref[...]
    dsrc = dv[0, 0] + dv[1, 0]
    ddst = dv[0, 1] + dv[1, 1]
    so = lax.rsqrt(jnp.maximum(dsrc, 1.0))
    di = lax.rsqrt(jnp.maximum(ddst, 1.0))
    so_ref[...] = so
    di_ref[...] = di
    xs_ref[...] = x_ref[...] * so


def _tc_scale(x, deg):
    return pl.pallas_call(
        _scale_body,
        grid=(_NB,),
        in_specs=[
            pl.BlockSpec((_BLK, _D), lambda i: (i, 0)),
            pl.BlockSpec((_NC, 2, _BLK, 1), lambda i: (0, 0, i, 0)),
        ],
        out_specs=[
            pl.BlockSpec((_BLK, _D), lambda i: (i, 0)),
            pl.BlockSpec((_BLK, 1), lambda i: (i, 0)),
            pl.BlockSpec((_BLK, 1), lambda i: (i, 0)),
        ],
        out_shape=[
            jax.ShapeDtypeStruct((_NR, _D), jnp.float32),
            jax.ShapeDtypeStruct((_NR, 1), jnp.float32),
            jax.ShapeDtypeStruct((_NR, 1), jnp.float32),
        ],
    )(x, deg)


def _mm_body(p_ref, di_ref, w_ref, b_ref, u_ref, st_ref, acc_ref):
    i = pl.program_id(0)
    pv = p_ref[...]
    p = (pv[0] + pv[1]) * di_ref[...]
    u = jnp.dot(p, w_ref[...], preferred_element_type=jnp.float32) + b_ref[...]
    u_ref[...] = u
    rid = i * _BLK + lax.broadcasted_iota(jnp.int32, (_BLK, 1), 0)
    uz = jnp.where(rid < _N, u, 0.0)
    ssum = jnp.sum(uz, axis=0, keepdims=True)
    ssq = jnp.sum(uz * uz, axis=0, keepdims=True)
    st = jnp.concatenate(
        [ssum, ssq, jnp.zeros((6, _D), jnp.float32)], axis=0)

    @pl.when(i == 0)
    def _():
        acc_ref[...] = st

    @pl.when(i > 0)
    def _():
        acc_ref[...] = acc_ref[...] + st

    @pl.when(i == _NB - 1)
    def _():
        st_ref[...] = acc_ref[...]


def _tc_mm(parts, dinv, W, b):
    return pl.pallas_call(
        _mm_body,
        grid=(_NB,),
        in_specs=[
            pl.BlockSpec((_NC, _BLK, _D), lambda i: (0, i, 0)),
            pl.BlockSpec((_BLK, 1), lambda i: (i, 0)),
            pl.BlockSpec((_D, _D), lambda i: (0, 0)),
            pl.BlockSpec((1, _D), lambda i: (0, 0)),
        ],
        out_specs=[
            pl.BlockSpec((_BLK, _D), lambda i: (i, 0)),
            pl.BlockSpec((8, _D), lambda i: (0, 0)),
        ],
        out_shape=[
            jax.ShapeDtypeStruct((_NR, _D), jnp.float32),
            jax.ShapeDtypeStruct((8, _D), jnp.float32),
        ],
        scratch_shapes=[pltpu.VMEM((8, _D), jnp.float32)],
    )(parts, dinv, W, b)


def _bn_body(u_ref, st_ref, g_ref, bt_ref, so_ref, h_ref, hs_ref):
    mu = st_ref[0:1, :] * (1.0 / _N)
    m2 = st_ref[1:2, :] * (1.0 / _N)
    var = m2 - mu * mu
    inv = lax.rsqrt(var + 1e-5)
    h = (u_ref[...] - mu) * inv * g_ref[...] + bt_ref[...]
    h = jnp.maximum(h, 0.0)
    h_ref[...] = h
    hs_ref[...] = h * so_ref[...]


def _tc_bn(u, st, g, bt, sout):
    return pl.pallas_call(
        _bn_body,
        grid=(_NB,),
        in_specs=[
            pl.BlockSpec((_BLK, _D), lambda i: (i, 0)),
            pl.BlockSpec((8, _D), lambda i: (0, 0)),
            pl.BlockSpec((1, _D), lambda i: (0, 0)),
            pl.BlockSpec((1, _D), lambda i: (0, 0)),
            pl.BlockSpec((_BLK, 1), lambda i: (i, 0)),
        ],
        out_specs=[
            pl.BlockSpec((_BLK, _D), lambda i: (i, 0)),
            pl.BlockSpec((_BLK, _D), lambda i: (i, 0)),
        ],
        out_shape=[
            jax.ShapeDtypeStruct((_NR, _D), jnp.float32),
            jax.ShapeDtypeStruct((_NR, _D), jnp.float32),
        ],
    )(u, st, g, bt, sout)


def _sort_row(row):
    ones = jnp.ones((1, _D), jnp.float32)
    vi = lax.dot_general(row, ones, (((0,), (0,)), ((), ())),
                         preferred_element_type=jnp.float32)
    vj = jnp.broadcast_to(row, (_D, _D))
    ii = lax.broadcasted_iota(jnp.int32, (_D, _D), 0)
    jj = lax.broadcasted_iota(jnp.int32, (_D, _D), 1)
    before = (vj < vi) | ((vj == vi) & (jj < ii))
    rank = jnp.sum(before.astype(jnp.float32), axis=1, keepdims=True)
    onehot = (rank == jj.astype(jnp.float32)).astype(jnp.float32)
    return jnp.dot(row, onehot, preferred_element_type=jnp.float32)


def _pool_body(x_ref, h1_ref, h2_ref, p0_ref, q0_ref, p1_ref, q1_ref,
               p2_ref, q2_ref, o_ref):
    total = q0_ref[...] + q1_ref[...] + q2_ref[...]
    for h_ref, rows, p_ref in ((x_ref, _N, p0_ref),
                               (h1_ref, _NR, p1_ref),
                               (h2_ref, _NR, p2_ref)):
        hv = h_ref[...]
        rid = lax.broadcasted_iota(jnp.int32, (rows, 1), 0)
        if rows != _N:
            hv = jnp.where(rid < _N, hv, 0.0)
        m = jnp.max(hv, axis=1, keepdims=True)
        if rows != _N:
            m = jnp.where(rid < _N, m, -jnp.inf)
        pv = p_ref[...]
        for t in range(_K):
            mx = jnp.max(m)
            idx = jnp.min(jnp.where(m == mx, rid, _N))
            sel = (rid == idx).astype(jnp.float32)
            row = lax.dot_general(sel, hv, (((0,), (0,)), ((), ())),
                                  preferred_element_type=jnp.float32)
            srow = _sort_row(row)
            total = total + jnp.dot(srow, pv[t * _D:(t + 1) * _D, :],
                                    preferred_element_type=jnp.float32)
            m = jnp.where(rid == idx, -jnp.inf, m)
    o_ref[...] = total


def _tc_pool(x, h1, h2, P0, pb0, P1, pb1, P2, pb2):
    return pl.pallas_call(
        _pool_body,
        out_shape=jax.ShapeDtypeStruct((1, pb0.shape[1]), jnp.float32),
    )(x, h1, h2, P0, pb0, P1, pb1, P2, pb2)


def kernel(x, edge_index, W1, b1, g1, bt1, W2, b2, g2, bt2,
           P0, pb0, P1, pb1, P2, pb2):
    src = edge_index[0].reshape(_NW, _EP)
    dst = edge_index[1].reshape(_NW, _EP)
    pad = jnp.full((_NW, _EPP - _EP), _N, jnp.int32)
    ep = jnp.stack([
        jnp.concatenate([src, pad], axis=1).reshape(_NW, _NCH, _CH),
        jnp.concatenate([dst, pad], axis=1).reshape(_NW, _NCH, _CH),
    ])

    sc_degrees, sc_segsum = _sc_kernels()
    deg = sc_degrees(ep).reshape(_NC, 2, 128 * 128, 1)[:, :, :_NR]
    xs, sout, dinv = _tc_scale(x, deg)
    parts1 = sc_segsum(ep, xs)
    u1, st1 = _tc_mm(parts1, dinv, W1, b1.reshape(1, -1))
    h1, h1s = _tc_bn(u1, st1, g1.reshape(1, -1), bt1.reshape(1, -1), sout)
    parts2 = sc_segsum(ep, h1s)
    u2, st2 = _tc_mm(parts2, dinv, W2, b2.reshape(1, -1))
    h2, _ = _tc_bn(u2, st2, g2.reshape(1, -1), bt2.reshape(1, -1), sout)
    return _tc_pool(x, h1, h2, P0, pb0.reshape(1, -1), P1, pb1.reshape(1, -1),
                    P2, pb2.reshape(1, -1))


# speedup vs baseline: 1.3734x; 1.3734x over previous
import functools

import jax
import jax.numpy as jnp
from jax import lax
from jax.experimental import pallas as pl
from jax.experimental.pallas import tpu as pltpu
from jax.experimental.pallas import tpu_sc as plsc

_N = 10000
_E = 320000
_D = 128
_K = 3

_NC = 2
_NS = 16
_NW = _NC * _NS
_EP = _E // _NW
_CH = 128
_NCH = -(-_EP // _CH)
_EPP = _NCH * _CH
_NR = 10240
_RPT = _NR // _NS

_BLK = 512
_NB = _NR // _BLK

@functools.cache
def _sc_kernels():
    mesh = plsc.VectorSubcoreMesh(
        core_axis_name="c", subcore_axis_name="s",
        num_cores=_NC, num_subcores=_NS)

    HR = 128
    RR = HR // _NS

    @functools.partial(
        pl.kernel,
        out_type=jax.ShapeDtypeStruct((_NC, 2, HR, 128), jnp.float32),
        mesh=mesh,
        compiler_params=pltpu.CompilerParams(needs_layout_passes=False),
        scratch_types=[
            pltpu.VMEM((_NCH, _CH), jnp.int32),
            pltpu.VMEM((_NCH, _CH), jnp.int32),
            pltpu.VMEM((HR, 128), jnp.float32),
            pltpu.VMEM((_NS, RR, 128), jnp.float32),
            pltpu.VMEM_SHARED((_NS, HR, 128), jnp.float32),
        ],
    )
    def sc_degrees(e_hbm, out_hbm, sidx, didx, hist, rbuf, part):
        c = lax.axis_index("c")
        s = lax.axis_index("s")
        w = c * _NS + s
        pltpu.sync_copy(e_hbm.at[0, w], sidx)
        pltpu.sync_copy(e_hbm.at[1, w], didx)
        ones = jnp.ones((16,), jnp.float32)
        for kind in range(2):
            idxbuf = sidx if kind == 0 else didx

            def zh(i, carry):
                for k16 in range(128 // 16):
                    hist[i, pl.ds(k16 * 16, 16)] = jnp.zeros((16,), jnp.float32)
                return carry

            lax.fori_loop(0, HR, zh, 0)

            def hloop(j, carry):
                for k16 in range(_CH // 16):
                    v = idxbuf[j, pl.ds(k16 * 16, 16)]
                    hi = jax.lax.shift_right_logical(v, 7)
                    lo = jax.lax.bitwise_and(v, 127)
                    plsc.addupdate_scatter(hist, [hi, lo], ones)
                return carry

            lax.fori_loop(0, _NCH, hloop, 0)

            pltpu.sync_copy(hist, part.at[s])
            plsc.subcore_barrier()
            for p in range(_NS):
                pltpu.sync_copy(part.at[p, pl.ds(s * RR, RR)], rbuf.at[p])

            def red(k, carry):
                for k16 in range(128 // 16):
                    acc = rbuf[0, k, pl.ds(k16 * 16, 16)]
                    for p in range(1, _NS):
                        acc = acc + rbuf[p, k, pl.ds(k16 * 16, 16)]
                    hist[k, pl.ds(k16 * 16, 16)] = acc
                return carry

            lax.fori_loop(0, RR, red, 0)
            pltpu.sync_copy(hist.at[pl.ds(0, RR)],
                            out_hbm.at[c, kind, pl.ds(s * RR, RR)])
            plsc.subcore_barrier()

    @functools.partial(
        pl.kernel,
        out_type=jax.ShapeDtypeStruct((_NC, _N, _D), jnp.float32),
        mesh=mesh,
        compiler_params=pltpu.CompilerParams(needs_layout_passes=False),
        scratch_types=[
            pltpu.VMEM((_NCH, _CH), jnp.int32),
            pltpu.VMEM((_NCH, _CH), jnp.int32),
            pltpu.VMEM((_CH, _D), jnp.float32),
            pltpu.VMEM_SHARED((_NR, _D), jnp.float32),
            pltpu.SemaphoreType.DMA,
        ],
    )
    def sc_segsum(e_hbm, xs_hbm, out_hbm, sidx, didx, gbuf, acc, sem):
        c = lax.axis_index("c")
        s = lax.axis_index("s")
        w = c * _NS + s

        def fill(i, carry):
            for k16 in range(_D // 16):
                gbuf[i, pl.ds(k16 * 16, 16)] = jnp.zeros((16,), jnp.float32)
            return carry

        lax.fori_loop(0, _CH, fill, 0)

        def zero(i, carry):
            pltpu.sync_copy(gbuf, acc.at[pl.ds(s * _RPT + i * _CH, _CH)])
            return carry

        lax.fori_loop(0, _RPT // _CH, zero, 0)

        pltpu.sync_copy(e_hbm.at[0, w], sidx)
        pltpu.sync_copy(e_hbm.at[1, w], didx)
        plsc.subcore_barrier()

        def step(j, carry):
            pltpu.async_copy(xs_hbm.at[sidx.at[j]], gbuf, sem).wait()
            pltpu.sync_copy(gbuf, acc.at[didx.at[j]], add=True)
            return carry

        lax.fori_loop(0, _NCH, step, 0)
        plsc.subcore_barrier()

        @pl.when(s < _NS - 1)
        def _():
            pltpu.sync_copy(acc.at[pl.ds(s * _RPT, _RPT)],
                            out_hbm.at[c, pl.ds(s * _RPT, _RPT)])

        @pl.when(s == _NS - 1)
        def _():
            tail = _N - (_NS - 1) * _RPT
            pltpu.sync_copy(acc.at[pl.ds((_NS - 1) * _RPT, tail)],
                            out_hbm.at[c, pl.ds((_NS - 1) * _RPT, tail)])

    return sc_degrees, sc_segsum


def _scale_body(x_ref, d_ref, xs_ref, so_ref, di_ref):
    dv = d_ref[...]
    dsrc = dv[0, 0] + dv[1, 0]
    ddst = dv[0, 1] + dv[1, 1]
    so = lax.rsqrt(jnp.maximum(dsrc, 1.0))
    di = lax.rsqrt(jnp.maximum(ddst, 1.0))
    so_ref[...] = so
    di_ref[...] = di
    xs_ref[...] = x_ref[...] * so


def _tc_scale(x, deg):
    return pl.pallas_call(
        _scale_body,
        grid=(_NB,),
        in_specs=[
            pl.BlockSpec((_BLK, _D), lambda i: (i, 0)),
            pl.BlockSpec((_NC, 2, _BLK, 1), lambda i: (0, 0, i, 0)),
        ],
        out_specs=[
            pl.BlockSpec((_BLK, _D), lambda i: (i, 0)),
            pl.BlockSpec((_BLK, 1), lambda i: (i, 0)),
            pl.BlockSpec((_BLK, 1), lambda i: (i, 0)),
        ],
        out_shape=[
            jax.ShapeDtypeStruct((_NR, _D), jnp.float32),
            jax.ShapeDtypeStruct((_NR, 1), jnp.float32),
            jax.ShapeDtypeStruct((_NR, 1), jnp.float32),
        ],
    )(x, deg)


def _mm_body(p_ref, di_ref, w_ref, b_ref, u_ref, st_ref, acc_ref):
    i = pl.program_id(0)
    pv = p_ref[...]
    p = (pv[0] + pv[1]) * di_ref[...]
    u = jnp.dot(p, w_ref[...], preferred_element_type=jnp.float32) + b_ref[...]
    u_ref[...] = u
    rid = i * _BLK + lax.broadcasted_iota(jnp.int32, (_BLK, 1), 0)
    uz = jnp.where(rid < _N, u, 0.0)
    ssum = jnp.sum(uz, axis=0, keepdims=True)
    ssq = jnp.sum(uz * uz, axis=0, keepdims=True)
    st = jnp.concatenate(
        [ssum, ssq, jnp.zeros((6, _D), jnp.float32)], axis=0)

    @pl.when(i == 0)
    def _():
        acc_ref[...] = st

    @pl.when(i > 0)
    def _():
        acc_ref[...] = acc_ref[...] + st

    @pl.when(i == _NB - 1)
    def _():
        st_ref[...] = acc_ref[...]


def _tc_mm(parts, dinv, W, b):
    return pl.pallas_call(
        _mm_body,
        grid=(_NB,),
        in_specs=[
            pl.BlockSpec((_NC, _BLK, _D), lambda i: (0, i, 0)),
            pl.BlockSpec((_BLK, 1), lambda i: (i, 0)),
            pl.BlockSpec((_D, _D), lambda i: (0, 0)),
            pl.BlockSpec((1, _D), lambda i: (0, 0)),
        ],
        out_specs=[
            pl.BlockSpec((_BLK, _D), lambda i: (i, 0)),
            pl.BlockSpec((8, _D), lambda i: (0, 0)),
        ],
        out_shape=[
            jax.ShapeDtypeStruct((_NR, _D), jnp.float32),
            jax.ShapeDtypeStruct((8, _D), jnp.float32),
        ],
        scratch_shapes=[pltpu.VMEM((8, _D), jnp.float32)],
    )(parts, dinv, W, b)


def _bn_body(u_ref, st_ref, g_ref, bt_ref, so_ref, h_ref, hs_ref):
    mu = st_ref[0:1, :] * (1.0 / _N)
    m2 = st_ref[1:2, :] * (1.0 / _N)
    var = m2 - mu * mu
    inv = lax.rsqrt(var + 1e-5)
    h = (u_ref[...] - mu) * inv * g_ref[...] + bt_ref[...]
    h = jnp.maximum(h, 0.0)
    h_ref[...] = h
    hs_ref[...] = h * so_ref[...]


def _tc_bn(u, st, g, bt, sout):
    return pl.pallas_call(
        _bn_body,
        grid=(_NB,),
        in_specs=[
            pl.BlockSpec((_BLK, _D), lambda i: (i, 0)),
            pl.BlockSpec((8, _D), lambda i: (0, 0)),
            pl.BlockSpec((1, _D), lambda i: (0, 0)),
            pl.BlockSpec((1, _D), lambda i: (0, 0)),
            pl.BlockSpec((_BLK, 1), lambda i: (i, 0)),
        ],
        out_specs=[
            pl.BlockSpec((_BLK, _D), lambda i: (i, 0)),
            pl.BlockSpec((_BLK, _D), lambda i: (i, 0)),
        ],
        out_shape=[
            jax.ShapeDtypeStruct((_NR, _D), jnp.float32),
            jax.ShapeDtypeStruct((_NR, _D), jnp.float32),
        ],
    )(u, st, g, bt, sout)


def _sort_row(row):
    ones = jnp.ones((1, _D), jnp.float32)
    vi = lax.dot_general(row, ones, (((0,), (0,)), ((), ())),
                         preferred_element_type=jnp.float32)
    vj = jnp.broadcast_to(row, (_D, _D))
    ii = lax.broadcasted_iota(jnp.int32, (_D, _D), 0)
    jj = lax.broadcasted_iota(jnp.int32, (_D, _D), 1)
    before = (vj < vi) | ((vj == vi) & (jj < ii))
    rank = jnp.sum(before.astype(jnp.float32), axis=1, keepdims=True)
    onehot = (rank == jj.astype(jnp.float32)).astype(jnp.float32)
    return jnp.dot(row, onehot, preferred_element_type=jnp.float32)


def _pool_body(x_ref, h1_ref, h2_ref, p0_ref, q0_ref, p1_ref, q1_ref,
               p2_ref, q2_ref, o_ref):
    total = q0_ref[...] + q1_ref[...] + q2_ref[...]
    for h_ref, rows, p_ref in ((x_ref, _N, p0_ref),
                               (h1_ref, _NR, p1_ref),
                               (h2_ref, _NR, p2_ref)):
        hv = h_ref[...]
        rid = lax.broadcasted_iota(jnp.int32, (rows, 1), 0)
        if rows != _N:
            hv = jnp.where(rid < _N, hv, 0.0)
        m = jnp.max(hv, axis=1, keepdims=True)
        if rows != _N:
            m = jnp.where(rid < _N, m, -jnp.inf)
        pv = p_ref[...]
        for t in range(_K):
            mx = jnp.max(m)
            idx = jnp.min(jnp.where(m == mx, rid, _N))
            sel = (rid == idx).astype(jnp.float32)
            row = lax.dot_general(sel, hv, (((0,), (0,)), ((), ())),
                                  preferred_element_type=jnp.float32)
            srow = _sort_row(row)
            total = total + jnp.dot(srow, pv[t * _D:(t + 1) * _D, :],
                                    preferred_element_type=jnp.float32)
            m = jnp.where(rid == idx, -jnp.inf, m)
    o_ref[...] = total


def _tc_pool(x, h1, h2, P0, pb0, P1, pb1, P2, pb2):
    return pl.pallas_call(
        _pool_body,
        out_shape=jax.ShapeDtypeStruct((1, pb0.shape[1]), jnp.float32),
    )(x, h1, h2, P0, pb0, P1, pb1, P2, pb2)


def kernel(x, edge_index, W1, b1, g1, bt1, W2, b2, g2, bt2,
           P0, pb0, P1, pb1, P2, pb2):
    src = edge_index[0].reshape(_NW, _EP)
    dst = edge_index[1].reshape(_NW, _EP)
    pad = jnp.full((_NW, _EPP - _EP), _N, jnp.int32)
    ep = jnp.stack([
        jnp.concatenate([src, pad], axis=1).reshape(_NW, _NCH, _CH),
        jnp.concatenate([dst, pad], axis=1).reshape(_NW, _NCH, _CH),
    ])

    sc_degrees, sc_segsum = _sc_kernels()
    deg = sc_degrees(ep).reshape(_NC, 2, 128 * 128, 1)[:, :, :_NR]
    xs, sout, dinv = _tc_scale(x, deg)
    parts1 = sc_segsum(ep, xs)
    u1, st1 = _tc_mm(parts1, dinv, W1, b1.reshape(1, -1))
    h1, h1s = _tc_bn(u1, st1, g1.reshape(1, -1), bt1.reshape(1, -1), sout)
    parts2 = sc_segsum(ep, h1s)
    u2, st2 = _tc_mm(parts2, dinv, W2, b2.reshape(1, -1))
    h2, _ = _tc_bn(u2, st2, g2.reshape(1, -1), bt2.reshape(1, -1), sout)
    return _tc_pool(x, h1, h2, P0, pb0.reshape(1, -1), P1, pb1.reshape(1, -1),
                    P2, pb2.reshape(1, -1))
